# Initial kernel scaffold; baseline (speedup 1.0000x reference)
import jax
import jax.numpy as jnp
from jax.experimental import pallas as pl


def kernel(x, edge_index, W1, as1, ad1, bg1, Wr1, br1, gam1, bet1, W2, as2, ad2, bg2, Wr2, br2, gam2, bet2, W3, as3, ad3, bg3, Wr3, br3):
    raise NotImplementedError("write your pallas kernel here")


# speedup vs baseline: 44.5541x; 44.5541x over previous
import functools

import jax
import jax.numpy as jnp
from jax import lax
from jax.experimental import pallas as pl
from jax.experimental.pallas import tpu as pltpu
import jax.experimental.pallas.tpu_sc as plsc

N = 10000
E = 320000
D = 128
NHEAD = 8
HC = 16

NC = 2
NS = 16
NW = NC * NS
LB = 16

EB = 128
E_TOT = E + N
KBLK = -(-E_TOT // (NW * EB))
E_PAD = KBLK * NW * EB

ROWS_PER_TILE = N // NS
FLUSH_ROWS = (N // NS) // 8 * 8

NB = 1000
GRID = N // NB


def _att_mat(a):
    h, hc = a.shape
    A = jnp.zeros((D, LB), jnp.float32)
    rows = jnp.arange(D)
    cols = jnp.repeat(jnp.arange(h), hc)
    return A.at[rows, cols].set(a.reshape(-1))


def _den_bcast_mat(hc):
    return (jnp.arange(LB)[:, None] == (jnp.arange(D) // hc)[None, :]).astype(
        jnp.float32)


def _tc_first_body(x_ref, w_ref, wr_ref, as_ref, ad_ref,
                   xl_ref, sa_ref, da_ref, res_ref):
    xv = x_ref[...]
    xl = jnp.dot(xv, w_ref[...].T, preferred_element_type=jnp.float32)
    xl_ref[...] = xl
    sa_ref[...] = jnp.dot(xl, as_ref[...], preferred_element_type=jnp.float32)
    da_ref[...] = jnp.dot(xl, ad_ref[...], preferred_element_type=jnp.float32)
    res_ref[...] = jnp.dot(xv, wr_ref[...].T, preferred_element_type=jnp.float32)


def _tc_comb_body(np_ref, dp_ref, res_ref, bvec_ref, scale_ref, shift_ref,
                  r_ref, w_ref, wr_ref, as_ref, ad_ref,
                  xl_ref, sa_ref, da_ref, res2_ref):
    num = np_ref[0] + np_ref[1]
    den = dp_ref[0] + dp_ref[1]
    denb = jnp.dot(den, r_ref[...], preferred_element_type=jnp.float32) + 1e-16
    hv = num / denb + res_ref[...] + bvec_ref[...]
    hv = jnp.maximum(hv * scale_ref[...] + shift_ref[...], 0.0)
    xl = jnp.dot(hv, w_ref[...].T, preferred_element_type=jnp.float32)
    xl_ref[...] = xl
    sa_ref[...] = jnp.dot(xl, as_ref[...], preferred_element_type=jnp.float32)
    da_ref[...] = jnp.dot(xl, ad_ref[...], preferred_element_type=jnp.float32)
    res2_ref[...] = jnp.dot(hv, wr_ref[...].T, preferred_element_type=jnp.float32)


def _tc_final_body(np_ref, dp_ref, res_ref, bvec_ref, r_ref, out_ref):
    num = np_ref[0] + np_ref[1]
    den = dp_ref[0] + dp_ref[1]
    denb = jnp.dot(den, r_ref[...], preferred_element_type=jnp.float32) + 1e-16
    out_ref[...] = num / denb + res_ref[...] + bvec_ref[...]


_FULL = lambda shape: pl.BlockSpec(shape, lambda i: tuple(0 for _ in shape))
_ROWS = pl.BlockSpec((NB, D), lambda i: (i, 0))
_ROWS16 = pl.BlockSpec((NB, LB), lambda i: (i, 0))
_PARTS = pl.BlockSpec((NC, NB, D), lambda i: (0, i, 0))
_PARTS16 = pl.BlockSpec((NC, NB, LB), lambda i: (0, i, 0))


def _tc_first(x, w, wr, a_s, a_d):
    return pl.pallas_call(
        _tc_first_body,
        grid=(GRID,),
        in_specs=[_ROWS, _FULL((D, D)), _FULL((D, D)),
                  _FULL((D, LB)), _FULL((D, LB))],
        out_specs=[_ROWS, _ROWS16, _ROWS16, _ROWS],
        out_shape=[jax.ShapeDtypeStruct((N, D), jnp.float32),
                   jax.ShapeDtypeStruct((N, LB), jnp.float32),
                   jax.ShapeDtypeStruct((N, LB), jnp.float32),
                   jax.ShapeDtypeStruct((N, D), jnp.float32)],
    )(x, w, wr, a_s, a_d)


def _tc_comb(npart, dpart, res, bvec, scale, shift, r, w, wr, a_s, a_d):
    return pl.pallas_call(
        _tc_comb_body,
        grid=(GRID,),
        in_specs=[_PARTS, _PARTS16, _ROWS, _FULL((1, D)), _FULL((1, D)),
                  _FULL((1, D)), _FULL((LB, D)), _FULL((D, D)), _FULL((D, D)),
                  _FULL((D, LB)), _FULL((D, LB))],
        out_specs=[_ROWS, _ROWS16, _ROWS16, _ROWS],
        out_shape=[jax.ShapeDtypeStruct((N, D), jnp.float32),
                   jax.ShapeDtypeStruct((N, LB), jnp.float32),
                   jax.ShapeDtypeStruct((N, LB), jnp.float32),
                   jax.ShapeDtypeStruct((N, D), jnp.float32)],
    )(npart, dpart, res, bvec, scale, shift, r, w, wr, a_s, a_d)


def _tc_final(npart, dpart, res, bvec, r):
    return pl.pallas_call(
        _tc_final_body,
        grid=(GRID,),
        in_specs=[_PARTS, _PARTS16, _ROWS, _FULL((1, D)), _FULL((LB, D))],
        out_specs=_ROWS,
        out_shape=jax.ShapeDtypeStruct((N, D), jnp.float32),
    )(npart, dpart, res, bvec, r)


def _sc_edge_body(nh, src_hbm, dst_hbm, sa_hbm, da_hbm, xl_hbm,
                  np_out, dp_out,
                  src_i, dst_i, sa_v, da_v, xr_v, wp_v, zb_v, zbd_v,
                  num_acc, den_acc):
    cid = lax.axis_index("c")
    sid = lax.axis_index("s")
    wid = sid * NC + cid

    lane = lax.iota(jnp.int32, LB)
    z16 = lane.astype(jnp.float32) * 0.0
    maskc = jnp.minimum(jnp.maximum(nh - lane, 0), 1).astype(jnp.float32)

    def _z(i, _):
        zb_v[i // 8, pl.ds((i % 8) * LB, LB)] = z16
        return 0
    lax.fori_loop(0, 125 * 8, _z, 0)

    def _zd(i, _):
        zbd_v[i, :] = z16
        return 0
    lax.fori_loop(0, 125, _zd, 0)

    base_row = sid * ROWS_PER_TILE
    for r in range(5):
        pltpu.sync_copy(zb_v, num_acc.at[pl.ds(base_row + 125 * r, 125)])
        pltpu.sync_copy(zbd_v, den_acc.at[pl.ds(base_row + 125 * r, 125)])
    plsc.subcore_barrier()

    def _block(k, _):
        base = (k * NW + wid) * EB
        pltpu.sync_copy(src_hbm.at[pl.ds(base, EB)], src_i)
        pltpu.sync_copy(dst_hbm.at[pl.ds(base, EB)], dst_i)
        pltpu.sync_copy(sa_hbm.at[src_i], sa_v)
        pltpu.sync_copy(da_hbm.at[dst_i], da_v)
        pltpu.sync_copy(xl_hbm.at[src_i], xr_v)

        def _edge(e, _):
            v = sa_v[e, :] + da_v[e, :]
            v = jnp.where(v > 0, v, 0.2 * v)
            v = jnp.exp(v)
            valid_f = jnp.where(base + e < E_TOT, 1.0, 0.0)
            v = v * (maskc * valid_f)
            wp_v[e, :] = v
            for h in range(NHEAD):
                s = v[h % nh]
                xr_v[e, pl.ds(h * HC, HC)] = xr_v[e, pl.ds(h * HC, HC)] * s
            return 0
        lax.fori_loop(0, EB, _edge, 0)

        pltpu.sync_copy(xr_v, num_acc.at[dst_i], add=True)
        pltpu.sync_copy(wp_v, den_acc.at[dst_i], add=True)
        return 0

    lax.fori_loop(0, KBLK, _block, 0)
    plsc.subcore_barrier()

    rs = pl.ds(sid * FLUSH_ROWS, FLUSH_ROWS)
    pltpu.sync_copy(num_acc.at[rs], np_out.at[cid, rs])
    pltpu.sync_copy(den_acc.at[rs], dp_out.at[cid, rs])

    @pl.when(sid == NS - 1)
    def _tail():
        rs2 = pl.ds(NS * FLUSH_ROWS, N - NS * FLUSH_ROWS)
        pltpu.sync_copy(num_acc.at[rs2], np_out.at[cid, rs2])
        pltpu.sync_copy(den_acc.at[rs2], dp_out.at[cid, rs2])


def _sc_edge(nh):
    return pl.kernel(
        functools.partial(_sc_edge_body, nh),
        out_type=[jax.ShapeDtypeStruct((NC, N, D), jnp.float32),
                  jax.ShapeDtypeStruct((NC, N, LB), jnp.float32)],
        mesh=plsc.VectorSubcoreMesh(core_axis_name="c", subcore_axis_name="s",
                                    num_cores=NC, num_subcores=NS),
        compiler_params=pltpu.CompilerParams(use_tc_tiling_on_sc=False),
        scratch_types=[
            pltpu.VMEM((EB,), jnp.int32),
            pltpu.VMEM((EB,), jnp.int32),
            pltpu.VMEM((EB, LB), jnp.float32),
            pltpu.VMEM((EB, LB), jnp.float32),
            pltpu.VMEM((EB, D), jnp.float32),
            pltpu.VMEM((EB, LB), jnp.float32),
            pltpu.VMEM((125, D), jnp.float32),
            pltpu.VMEM((125, LB), jnp.float32),
            pltpu.VMEM_SHARED((N, D), jnp.float32),
            pltpu.VMEM_SHARED((N, LB), jnp.float32),
        ],
    )


def kernel(x, edge_index, W1, as1, ad1, bg1, Wr1, br1, gam1, bet1,
           W2, as2, ad2, bg2, Wr2, br2, gam2, bet2,
           W3, as3, ad3, bg3, Wr3, br3):
    loop = jnp.arange(N, dtype=jnp.int32)
    pad = jnp.zeros((E_PAD - E_TOT,), jnp.int32)
    src = jnp.concatenate([edge_index[0], loop, pad])
    dst = jnp.concatenate([edge_index[1], loop, pad])

    bn_scale1 = (gam1 / jnp.sqrt(jnp.float32(1.0 + 1e-5))).reshape(1, D)
    bn_scale2 = (gam2 / jnp.sqrt(jnp.float32(1.0 + 1e-5))).reshape(1, D)
    bv1 = (bg1 + br1).reshape(1, D)
    bv2 = (bg2 + br2).reshape(1, D)
    bv3 = (bg3 + br3).reshape(1, D)
    r8 = _den_bcast_mat(HC)
    r1 = _den_bcast_mat(D)

    edge8 = _sc_edge(NHEAD)
    edge1 = _sc_edge(1)

    xl1, sa1, da1, res1 = _tc_first(x, W1, Wr1, _att_mat(as1), _att_mat(ad1))
    np1, dp1 = edge8(src, dst, sa1, da1, xl1)
    xl2, sa2, da2, res2 = _tc_comb(np1, dp1, res1, bv1, bn_scale1,
                                   bet1.reshape(1, D), r8, W2, Wr2,
                                   _att_mat(as2), _att_mat(ad2))
    np2, dp2 = edge8(src, dst, sa2, da2, xl2)
    xl3, sa3, da3, res3 = _tc_comb(np2, dp2, res2, bv2, bn_scale2,
                                   bet2.reshape(1, D), r8, W3, Wr3,
                                   _att_mat(as3), _att_mat(ad3))
    np3, dp3 = edge1(src, dst, sa3, da3, xl3)
    return _tc_final(np3, dp3, res3, bv3, r1)


# speedup vs baseline: 83.1826x; 1.8670x over previous
import functools

import jax
import jax.numpy as jnp
from jax import lax
from jax.experimental import pallas as pl
from jax.experimental.pallas import tpu as pltpu
import jax.experimental.pallas.tpu_sc as plsc

N = 10000
E = 320000
D = 128
NHEAD = 8
HC = 16

NC = 2
NS = 16
NW = NC * NS
LB = 16

EB = 64
E_TOT = E + N
NBUF = 3
CH = 6
KBLK = 162
NCHUNK = KBLK // CH
E_PAD = KBLK * NW * EB
N_ACC = N + 8

ROWS_PER_TILE = N // NS
FLUSH_ROWS = (N // NS) // 8 * 8

NB = 1000
GRID = N // NB


def _att_mat(a):
    h, hc = a.shape
    if h == 1:
        return jnp.pad(jnp.tile(a.reshape(D, 1), (1, NHEAD)),
                       ((0, 0), (0, LB - NHEAD)))
    A = jnp.zeros((D, LB), jnp.float32)
    rows = jnp.arange(D)
    cols = jnp.repeat(jnp.arange(h), hc)
    return A.at[rows, cols].set(a.reshape(-1))


def _den_bcast_mat(hc):
    return (jnp.arange(LB)[:, None] == (jnp.arange(D) // hc)[None, :]).astype(
        jnp.float32)


def _tc_first_body(x_ref, w_ref, wr_ref, as_ref, ad_ref,
                   xl_ref, sa_ref, da_ref, res_ref):
    xv = x_ref[...]
    xl = jnp.dot(xv, w_ref[...].T, preferred_element_type=jnp.float32)
    xl_ref[...] = xl
    sa_ref[...] = jnp.dot(xl, as_ref[...], preferred_element_type=jnp.float32)
    da_ref[...] = jnp.dot(xl, ad_ref[...], preferred_element_type=jnp.float32)
    res_ref[...] = jnp.dot(xv, wr_ref[...].T, preferred_element_type=jnp.float32)


def _tc_comb_body(np_ref, dp_ref, res_ref, bvec_ref, scale_ref, shift_ref,
                  r_ref, w_ref, wr_ref, as_ref, ad_ref,
                  xl_ref, sa_ref, da_ref, res2_ref):
    num = np_ref[0] + np_ref[1]
    den = dp_ref[0] + dp_ref[1]
    denb = jnp.dot(den, r_ref[...], preferred_element_type=jnp.float32) + 1e-16
    hv = num / denb + res_ref[...] + bvec_ref[...]
    hv = jnp.maximum(hv * scale_ref[...] + shift_ref[...], 0.0)
    xl = jnp.dot(hv, w_ref[...].T, preferred_element_type=jnp.float32)
    xl_ref[...] = xl
    sa_ref[...] = jnp.dot(xl, as_ref[...], preferred_element_type=jnp.float32)
    da_ref[...] = jnp.dot(xl, ad_ref[...], preferred_element_type=jnp.float32)
    res2_ref[...] = jnp.dot(hv, wr_ref[...].T, preferred_element_type=jnp.float32)


def _tc_final_body(np_ref, dp_ref, res_ref, bvec_ref, r_ref, out_ref):
    num = np_ref[0] + np_ref[1]
    den = dp_ref[0] + dp_ref[1]
    denb = jnp.dot(den, r_ref[...], preferred_element_type=jnp.float32) + 1e-16
    out_ref[...] = num / denb + res_ref[...] + bvec_ref[...]


_FULL = lambda shape: pl.BlockSpec(shape, lambda i: tuple(0 for _ in shape))
_ROWS = pl.BlockSpec((NB, D), lambda i: (i, 0))
_ROWS16 = pl.BlockSpec((NB, LB), lambda i: (i, 0))
_PARTS = pl.BlockSpec((NC, NB, D), lambda i: (0, i, 0))
_PARTS16 = pl.BlockSpec((NC, NB, LB), lambda i: (0, i, 0))


def _tc_first(x, w, wr, a_s, a_d):
    return pl.pallas_call(
        _tc_first_body,
        grid=(GRID,),
        in_specs=[_ROWS, _FULL((D, D)), _FULL((D, D)),
                  _FULL((D, LB)), _FULL((D, LB))],
        out_specs=[_ROWS, _ROWS16, _ROWS16, _ROWS],
        out_shape=[jax.ShapeDtypeStruct((N, D), jnp.float32),
                   jax.ShapeDtypeStruct((N, LB), jnp.float32),
                   jax.ShapeDtypeStruct((N, LB), jnp.float32),
                   jax.ShapeDtypeStruct((N, D), jnp.float32)],
    )(x, w, wr, a_s, a_d)


def _tc_comb(npart, dpart, res, bvec, scale, shift, r, w, wr, a_s, a_d):
    return pl.pallas_call(
        _tc_comb_body,
        grid=(GRID,),
        in_specs=[_PARTS, _PARTS16, _ROWS, _FULL((1, D)), _FULL((1, D)),
                  _FULL((1, D)), _FULL((LB, D)), _FULL((D, D)), _FULL((D, D)),
                  _FULL((D, LB)), _FULL((D, LB))],
        out_specs=[_ROWS, _ROWS16, _ROWS16, _ROWS],
        out_shape=[jax.ShapeDtypeStruct((N, D), jnp.float32),
                   jax.ShapeDtypeStruct((N, LB), jnp.float32),
                   jax.ShapeDtypeStruct((N, LB), jnp.float32),
                   jax.ShapeDtypeStruct((N, D), jnp.float32)],
    )(npart, dpart, res, bvec, scale, shift, r, w, wr, a_s, a_d)


def _tc_final(npart, dpart, res, bvec, r):
    return pl.pallas_call(
        _tc_final_body,
        grid=(GRID,),
        in_specs=[_PARTS, _PARTS16, _ROWS, _FULL((1, D)), _FULL((LB, D))],
        out_specs=_ROWS,
        out_shape=jax.ShapeDtypeStruct((N, D), jnp.float32),
    )(npart, dpart, res, bvec, r)


def _sc_edge_body(eidx_hbm, sa_hbm, da_hbm, xl_hbm,
                  np_out, dp_out,
                  idx_v, sa_v, da_v, xr_v, wp_v,
                  semg0, semg1, semg2, sems0, sems1, sems2, semi,
                  num_acc, den_acc):
    cid = lax.axis_index("c")
    sid = lax.axis_index("s")
    wid = sid * NC + cid
    semg = (semg0, semg1, semg2)
    sems = (sems0, sems1, sems2)

    z16 = lax.iota(jnp.int32, LB).astype(jnp.float32) * 0.0

    for b in range(NBUF):
        def _zx(i, _, b=b):
            xr_v[b, i // 8, pl.ds((i % 8) * LB, LB)] = z16
            return 0
        lax.fori_loop(0, EB * 8, _zx, 0)

        def _zw(i, _, b=b):
            wp_v[b, i, :] = z16
            return 0
        lax.fori_loop(0, EB, _zw, 0)

    base_row = sid * ROWS_PER_TILE
    for r in range(9):
        pltpu.sync_copy(xr_v.at[0], num_acc.at[pl.ds(base_row + EB * r, EB)])
        pltpu.sync_copy(wp_v.at[0], den_acc.at[pl.ds(base_row + EB * r, EB)])
    rem = ROWS_PER_TILE - 9 * EB
    pltpu.sync_copy(xr_v.at[0, pl.ds(0, rem)],
                    num_acc.at[pl.ds(base_row + 9 * EB, rem)])
    pltpu.sync_copy(wp_v.at[0, pl.ds(0, rem)],
                    den_acc.at[pl.ds(base_row + 9 * EB, rem)])
    plsc.subcore_barrier()

    def _fetch_idx(c, p):
        pltpu.async_copy(eidx_hbm.at[wid, pl.ds(c * CH, CH)], idx_v.at[p],
                         semi)

    def _drain_idx(p):
        pltpu.make_async_copy(eidx_hbm.at[wid, pl.ds(0, CH)], idx_v.at[p],
                              semi).wait()

    def _issue_scat(b, p, j):
        pltpu.async_copy(xr_v.at[b], num_acc.at[idx_v.at[p, j, 1]],
                         sems[b], add=True)
        pltpu.async_copy(wp_v.at[b], den_acc.at[idx_v.at[p, j, 1]],
                         sems[b], add=True)

    def _drain_scat(b):
        pltpu.make_async_copy(xr_v.at[b], num_acc.at[idx_v.at[0, 0, 1]],
                              sems[b]).wait()
        pltpu.make_async_copy(wp_v.at[b], den_acc.at[idx_v.at[0, 0, 1]],
                              sems[b]).wait()

    def _issue_gath(b, p, j):
        pltpu.async_copy(sa_hbm.at[idx_v.at[p, j, 0]], sa_v.at[b], semg[b])
        pltpu.async_copy(da_hbm.at[idx_v.at[p, j, 1]], da_v.at[b], semg[b])
        pltpu.async_copy(xl_hbm.at[idx_v.at[p, j, 0]], xr_v.at[b], semg[b])

    def _drain_gath(b):
        pltpu.make_async_copy(sa_hbm.at[idx_v.at[0, 0, 0]], sa_v.at[b],
                              semg[b]).wait()
        pltpu.make_async_copy(da_hbm.at[idx_v.at[0, 0, 1]], da_v.at[b],
                              semg[b]).wait()
        pltpu.make_async_copy(xl_hbm.at[idx_v.at[0, 0, 0]], xr_v.at[b],
                              semg[b]).wait()

    def _compute(b):
        def _edge(e, _):
            v = sa_v[b, e, :] + da_v[b, e, :]
            v = jnp.maximum(v, 0.2 * v)
            v = jnp.exp(v)
            wp_v[b, e, :] = v
            for h in range(NHEAD):
                s = v[h]
                xr_v[b, e, pl.ds(h * HC, HC)] = xr_v[b, e, pl.ds(h * HC, HC)] * s
            return 0
        lax.fori_loop(0, EB, _edge, 0)

    _fetch_idx(0, 0)
    _drain_idx(0)
    for b in range(NBUF):
        _issue_scat(b, 0, 0)
    _drain_scat(0)
    _issue_gath(0, 0, 0)
    _drain_scat(1)
    _issue_gath(1, 0, 1)

    def _chunk(c, _):
        p = lax.rem(c, 2)
        pn = lax.rem(c + 1, 2)
        cn = jnp.minimum(c + 1, NCHUNK - 1)
        for j in range(CH):
            b = j % NBUF
            _drain_gath(b)
            _compute(b)
            _issue_scat(b, p, j)
            if j == 1:
                _fetch_idx(cn, pn)
            if j == 3:
                _drain_idx(pn)
            bp = (b + 2) % NBUF
            _drain_scat(bp)
            if j < CH - 2:
                _issue_gath(bp, p, j + 2)
            else:
                _issue_gath(bp, pn, j + 2 - CH)
        return 0
    lax.fori_loop(0, NCHUNK, _chunk, 0)

    _drain_scat(NBUF - 1)
    _drain_gath(0)
    _drain_gath(1)
    plsc.subcore_barrier()

    rs = pl.ds(sid * FLUSH_ROWS, FLUSH_ROWS)
    pltpu.sync_copy(num_acc.at[rs], np_out.at[cid, rs])
    pltpu.sync_copy(den_acc.at[rs], dp_out.at[cid, rs])

    @pl.when(sid == NS - 1)
    def _tail():
        rs2 = pl.ds(NS * FLUSH_ROWS, N - NS * FLUSH_ROWS)
        pltpu.sync_copy(num_acc.at[rs2], np_out.at[cid, rs2])
        pltpu.sync_copy(den_acc.at[rs2], dp_out.at[cid, rs2])


def _sc_edge():
    return pl.kernel(
        _sc_edge_body,
        out_type=[jax.ShapeDtypeStruct((NC, N, D), jnp.float32),
                  jax.ShapeDtypeStruct((NC, N, LB), jnp.float32)],
        mesh=plsc.VectorSubcoreMesh(core_axis_name="c", subcore_axis_name="s",
                                    num_cores=NC, num_subcores=NS),
        compiler_params=pltpu.CompilerParams(use_tc_tiling_on_sc=False),
        scratch_types=[
            pltpu.VMEM((2, CH, 2, EB), jnp.int32),
            pltpu.VMEM((NBUF, EB, LB), jnp.float32),
            pltpu.VMEM((NBUF, EB, LB), jnp.float32),
            pltpu.VMEM((NBUF, EB, D), jnp.float32),
            pltpu.VMEM((NBUF, EB, LB), jnp.float32),
            pltpu.SemaphoreType.DMA,
            pltpu.SemaphoreType.DMA,
            pltpu.SemaphoreType.DMA,
            pltpu.SemaphoreType.DMA,
            pltpu.SemaphoreType.DMA,
            pltpu.SemaphoreType.DMA,
            pltpu.SemaphoreType.DMA,
            pltpu.VMEM_SHARED((N_ACC, D), jnp.float32),
            pltpu.VMEM_SHARED((N_ACC, LB), jnp.float32),
        ],
    )


def kernel(x, edge_index, W1, as1, ad1, bg1, Wr1, br1, gam1, bet1,
           W2, as2, ad2, bg2, Wr2, br2, gam2, bet2,
           W3, as3, ad3, bg3, Wr3, br3):
    loop = jnp.arange(N, dtype=jnp.int32)
    pad_s = jnp.zeros((E_PAD - E_TOT,), jnp.int32)
    pad_d = jnp.full((E_PAD - E_TOT,), N, jnp.int32)
    src = jnp.concatenate([edge_index[0], loop, pad_s])
    dst = jnp.concatenate([edge_index[1], loop, pad_d])
    eidx = jnp.stack([src, dst]).reshape(2, KBLK, NW, EB).transpose(2, 1, 0, 3)

    bn_scale1 = (gam1 / jnp.sqrt(jnp.float32(1.0 + 1e-5))).reshape(1, D)
    bn_scale2 = (gam2 / jnp.sqrt(jnp.float32(1.0 + 1e-5))).reshape(1, D)
    bv1 = (bg1 + br1).reshape(1, D)
    bv2 = (bg2 + br2).reshape(1, D)
    bv3 = (bg3 + br3).reshape(1, D)
    r8 = _den_bcast_mat(HC)
    r1 = _den_bcast_mat(D)

    edge = _sc_edge()

    xl1, sa1, da1, res1 = _tc_first(x, W1, Wr1, _att_mat(as1), _att_mat(ad1))
    np1, dp1 = edge(eidx, sa1, da1, xl1)
    xl2, sa2, da2, res2 = _tc_comb(np1, dp1, res1, bv1, bn_scale1,
                                   bet1.reshape(1, D), r8, W2, Wr2,
                                   _att_mat(as2), _att_mat(ad2))
    np2, dp2 = edge(eidx, sa2, da2, xl2)
    xl3, sa3, da3, res3 = _tc_comb(np2, dp2, res2, bv2, bn_scale2,
                                   bet2.reshape(1, D), r8, W3, Wr3,
                                   _att_mat(as3), _att_mat(ad3))
    np3, dp3 = edge(eidx, sa3, da3, xl3)
    return _tc_final(np3, dp3, res3, bv3, r1)
